# baseline (speedup 1.0000x reference)
import jax
import jax.numpy as jnp
from jax.experimental import pallas as pl
from jax.experimental.pallas import tpu as pltpu

N = 4096
BI = 512
BJ = 512
NI = N // BI
NJ = N // BJ
F1 = 256
H = 128
LSTM_PER_STEP = 2


def _lstm_steps(p_scr, whh_ref, h, c, t0, nsteps):
    for k in range(nsteps):
        g = p_scr[pl.ds(t0 + k, 1), :] + jax.lax.dot_general(
            h, whh_ref[...], (((1,), (1,)), ((), ())),
            preferred_element_type=jnp.float32)
        ig = jax.nn.sigmoid(g[:, 0:H])
        fg = jax.nn.sigmoid(g[:, H:2 * H])
        gg = jnp.tanh(g[:, 2 * H:3 * H])
        og = jax.nn.sigmoid(g[:, 3 * H:4 * H])
        c = fg * c + ig * gg
        h = og * jnp.tanh(c)
    return h, c


def _mega_body(adj_ref, x_ref, wih_ref, whh_ref, biasv_ref, w1t_ref,
               w2t_ref, b1_ref, b2_ref, w2_ref, b2r_ref, w3_ref, b3_ref,
               w4_ref, b4_ref, out_ref,
               mask_scr, deg_scr, ys1_scr, ys2_scr, s_scr,
               p_scr, h_scr, c_scr):
    jb = pl.program_id(0)
    i = pl.program_id(1)
    step_idx = jb * NI + i
    f32, bf16 = jnp.float32, jnp.bfloat16

    m = adj_ref[...] == 1
    mb = m.astype(bf16)
    mask_scr[jb, pl.ds(i * BI, BI), :] = mb
    part = jnp.sum(mb, axis=0, keepdims=True)

    @pl.when(i == 0)
    def _():
        deg_scr[jb] = part.astype(f32)

    @pl.when(i > 0)
    def _():
        deg_scr[jb] += part.astype(f32)

    @pl.when(step_idx == 0)
    def _():
        wb = wih_ref[...].astype(bf16)
        p_scr[...] = jax.lax.dot_general(
            x_ref[...], wb, (((0,), (1,)), ((), ())),
            preferred_element_type=f32) + biasv_ref[...]
        h_scr[...] = jnp.zeros((1, H), f32)
        c_scr[...] = jnp.zeros((1, H), f32)

    @pl.when(step_idx > 0)
    def _():
        t0 = (step_idx - 1) * LSTM_PER_STEP
        h, c = _lstm_steps(p_scr, whh_ref, h_scr[...], c_scr[...], t0,
                           LSTM_PER_STEP)
        h_scr[...] = h
        c_scr[...] = c

    @pl.when((jb == NJ - 1) & (i == NI - 1))
    def _():
        h, _ = _lstm_steps(p_scr, whh_ref, h_scr[...], c_scr[...],
                           (NJ * NI - 1) * LSTM_PER_STEP, LSTM_PER_STEP)

        for b in range(NJ):
            deg_scr[b] = jax.lax.rsqrt(deg_scr[b] + 1.0)

        for c in range(NI):
            dch = deg_scr[c]
            t = jax.lax.dot_general(
                w1t_ref[...], x_ref[c * BI:(c + 1) * BI, :],
                (((1,), (1,)), ((), ())), preferred_element_type=f32)
            ys1_scr[:, c * BI:(c + 1) * BI] = (t * dch).astype(bf16)

        for b in range(NJ):
            dj = deg_scr[b]
            contrib = jax.lax.dot_general(
                ys1_scr[...], mask_scr[b], (((1,), (0,)), ((), ())),
                preferred_element_type=f32)
            sp = jnp.sum(mask_scr[b] * dj.astype(bf16), axis=1,
                         keepdims=True).astype(f32)
            if b == 0:
                s_scr[...] = sp
            else:
                s_scr[...] += sp
            ysj = ys1_scr[:, b * BJ:(b + 1) * BJ]
            z = jnp.maximum(
                dj * contrib + dj * ysj.astype(f32) + b1_ref[...], 0.0)
            y2b = jax.lax.dot_general(
                w2t_ref[...], z.astype(bf16), (((1,), (0,)), ((), ())),
                preferred_element_type=f32) * dj
            ys2_scr[:, b * BJ:(b + 1) * BJ] = y2b.astype(bf16)

        gsum = jnp.zeros((F1, 1), f32)
        for b in range(NJ):
            dj = deg_scr[b]
            contrib = jax.lax.dot_general(
                ys2_scr[...], mask_scr[b], (((1,), (0,)), ((), ())),
                preferred_element_type=f32)
            ysj = ys2_scr[:, b * BJ:(b + 1) * BJ]
            x2 = jnp.maximum(
                dj * contrib + dj * ysj.astype(f32) + b2_ref[...], 0.0)
            s_b = s_scr[b * BJ:(b + 1) * BJ, :]
            d_col = jnp.reshape(dj, (BJ, 1))
            w = d_col * s_b + d_col * d_col
            gsum += jax.lax.dot_general(
                x2, w, (((1,), (0,)), ((), ())),
                preferred_element_type=f32)

        gsum_row = jnp.reshape(gsum, (1, F1))
        graph = jax.lax.dot_general(
            gsum_row, w2_ref[...], (((1,), (0,)), ((), ())),
            preferred_element_type=f32) * (1.0 / N) + b2r_ref[...]
        ev = jnp.maximum(
            jax.lax.dot_general(h, w3_ref[0:H, :], (((1,), (0,)), ((), ())),
                                preferred_element_type=f32)
            + jax.lax.dot_general(graph, w3_ref[H:H + F1, :],
                                  (((1,), (0,)), ((), ())),
                                  preferred_element_type=f32)
            + b3_ref[...], 0.0)
        cls = jax.lax.dot_general(
            ev, w4_ref[...], (((1,), (0,)), ((), ())),
            preferred_element_type=f32) + b4_ref[...]
        mx = jnp.max(cls, axis=1, keepdims=True)
        e = cls - mx
        out_ref[...] = e - jnp.log(jnp.sum(jnp.exp(e), axis=1,
                                           keepdims=True))


def kernel(adj_mat, node_att, W_ih, W_hh, b_ih, b_hh,
           W1, b1, W2, b2, W3, b3, W4, b4):
    f32 = jnp.float32
    bf16 = jnp.bfloat16
    w1t_bf = W1.T.astype(bf16)
    w2t_bf = W2.T.astype(bf16)

    out = pl.pallas_call(
        _mega_body,
        grid=(NJ, NI),
        in_specs=[
            pl.BlockSpec((BI, BJ), lambda j, i: (i, j)),
            pl.BlockSpec((N, H), lambda j, i: (0, 0)),
            pl.BlockSpec((4 * H, N), lambda j, i: (0, 0)),
            pl.BlockSpec((4 * H, H), lambda j, i: (0, 0)),
            pl.BlockSpec((1, 4 * H), lambda j, i: (0, 0)),
            pl.BlockSpec((F1, H), lambda j, i: (0, 0)),
            pl.BlockSpec((F1, F1), lambda j, i: (0, 0)),
            pl.BlockSpec((F1, 1), lambda j, i: (0, 0)),
            pl.BlockSpec((F1, 1), lambda j, i: (0, 0)),
            pl.BlockSpec((F1, F1), lambda j, i: (0, 0)),
            pl.BlockSpec((1, F1), lambda j, i: (0, 0)),
            pl.BlockSpec((H + F1, F1), lambda j, i: (0, 0)),
            pl.BlockSpec((1, F1), lambda j, i: (0, 0)),
            pl.BlockSpec((F1, 16), lambda j, i: (0, 0)),
            pl.BlockSpec((1, 16), lambda j, i: (0, 0)),
        ],
        out_specs=pl.BlockSpec((1, 16), lambda j, i: (0, 0)),
        out_shape=jax.ShapeDtypeStruct((1, 16), f32),
        scratch_shapes=[
            pltpu.VMEM((NJ, N, BJ), bf16),
            pltpu.VMEM((NJ, 1, BJ), f32),
            pltpu.VMEM((F1, N), bf16),
            pltpu.VMEM((F1, N), bf16),
            pltpu.VMEM((N, 1), f32),
            pltpu.VMEM((H, 4 * H), f32),
            pltpu.VMEM((1, H), f32),
            pltpu.VMEM((1, H), f32),
        ],
    )(adj_mat, node_att.astype(bf16), W_ih, W_hh,
      (b_ih + b_hh).reshape(1, 4 * H),
      w1t_bf, w2t_bf, b1.reshape(F1, 1), b2.reshape(F1, 1), W2,
      b2.reshape(1, F1), W3, b3.reshape(1, F1), W4, b4.reshape(1, 16))

    return out
